# baseline (speedup 1.0000x reference)
import jax
import jax.numpy as jnp
from jax import lax
from jax.experimental import pallas as pl
from jax.experimental.pallas import tpu as pltpu

_BN = 2048


def _fhs_body(surf_ref, emb_ref, o_ref):
    s = surf_ref[...]
    B, T = s.shape
    V = emb_ref.shape[0]
    oh = (s[:, :, None] == lax.broadcasted_iota(jnp.int32, (B, T, V), 2))
    counts = jnp.sum(oh.astype(jnp.float32), axis=1)
    o_ref[...] = jnp.dot(
        counts, emb_ref[...], preferred_element_type=jnp.float32) * (1.0 / T)


def _proj_body(fhs_ref, w_ref, b_ref, o_ref):
    i = pl.program_id(0)
    bias = b_ref[:, pl.ds(i * _BN, _BN)]
    o_ref[:, 0, :] = (
        jnp.dot(fhs_ref[...].astype(jnp.bfloat16),
                w_ref[...].astype(jnp.bfloat16),
                preferred_element_type=jnp.float32)
        + bias)


def kernel(surf, char_emb, root_codebook, suffix_W, suffix_b, suffix_codebook,
           ord_W, ord_b, ord_codebooks, out_W, out_b):
    B, T = surf.shape
    V, D = char_emb.shape
    _, N = out_W.shape
    nb = (N + _BN - 1) // _BN
    npad = nb * _BN
    b2d = jnp.pad(out_b, (0, npad - N)).reshape(1, npad)

    fhs = pl.pallas_call(
        _fhs_body,
        out_shape=jax.ShapeDtypeStruct((B, D), jnp.float32),
    )(surf, char_emb)

    out3d = pl.pallas_call(
        _proj_body,
        grid=(nb,),
        in_specs=[
            pl.BlockSpec((B, D), lambda i: (0, 0)),
            pl.BlockSpec((D, _BN), lambda i: (0, i)),
            pl.BlockSpec((1, npad), lambda i: (0, 0)),
        ],
        out_specs=pl.BlockSpec((B, 1, _BN), lambda i: (0, 0, i)),
        out_shape=jax.ShapeDtypeStruct((B, 1, N), jnp.float32),
        compiler_params=pltpu.CompilerParams(
            dimension_semantics=("arbitrary",)),
    )(fhs, out_W, b2d)
    return out3d
